# baseline (speedup 1.0000x reference)
import functools

import jax
import jax.numpy as jnp
from jax import lax
from jax.experimental import pallas as pl
from jax.experimental.pallas import tpu as pltpu
from jax.experimental.pallas import tpu_sc as plsc

_NC = 2
_NS = 16
_L = 16
_C = 80
_W = 144


def _pre_body(x_ref, wcat_ref, al_ref, ar_ref, whel_ref, er_ref, lmax_ref):
    x = x_ref[...]
    wh = jnp.dot(x, wcat_ref[...], preferred_element_type=jnp.float32)
    el = jnp.dot(wh, al_ref[...], preferred_element_type=jnp.float32)
    er = jnp.dot(wh, ar_ref[...], preferred_element_type=jnp.float32)
    nb = x.shape[0]
    z12 = jnp.zeros((nb, 12), jnp.float32)
    whel_ref[...] = jnp.concatenate([wh, el, z12], axis=1)
    er_ref[...] = jnp.concatenate([er, z12], axis=1)
    lmax = jnp.max(el, axis=0, keepdims=True)
    bm = jnp.concatenate([lmax, jnp.zeros((1, 124), jnp.float32)], axis=1)
    i = pl.program_id(0)

    @pl.when(i == 0)
    def _():
        lmax_ref[...] = bm

    @pl.when(i > 0)
    def _():
        lmax_ref[...] = jnp.maximum(lmax_ref[...], bm)


def _make_edge_kernel(n, e):
    ew_per = e // (_NC * _NS)
    chunks = ew_per // _C
    rows_per_tile = 632
    n_pad = rows_per_tile * _NS
    last_rows = n - (_NS - 1) * rows_per_tile

    mesh = plsc.VectorSubcoreMesh(core_axis_name="c", subcore_axis_name="s")

    @functools.partial(
        pl.kernel,
        out_type=jax.ShapeDtypeStruct((_NC, n, _W), jnp.float32),
        mesh=mesh,
        compiler_params=pltpu.CompilerParams(
            needs_layout_passes=False, use_tc_tiling_on_sc=False),
        scratch_types=[
            pltpu.VMEM((3, 3, _C), jnp.int32),
            pltpu.VMEM((3, _C), jnp.int32),
            pltpu.VMEM((3, _C, _L), jnp.float32),
            pltpu.VMEM((3, _C, _W), jnp.float32),
            pltpu.VMEM((1, 128), jnp.float32),
            pltpu.VMEM_SHARED((n_pad, _W), jnp.float32),
            pltpu.SemaphoreType.DMA,
            pltpu.SemaphoreType.DMA,
            pltpu.SemaphoreType.DMA,
            pltpu.SemaphoreType.DMA,
            pltpu.SemaphoreType.DMA,
            pltpu.SemaphoreType.DMA,
            pltpu.SemaphoreType.DMA,
            pltpu.SemaphoreType.DMA,
            pltpu.SemaphoreType.DMA,
        ],
    )
    def edge_kernel(rce_h, er_h, lmax_h, whel_h, acc_o,
                    rce_v, colS_v, er_v, whel_v, lmax_v, acc_sh,
                    isem0, isem1, isem2, gsem0, gsem1, gsem2,
                    ssem0, ssem1, ssem2):
        c = lax.axis_index("c")
        s = lax.axis_index("s")
        wid = c * _NS + s
        base = wid * ew_per
        r0 = s * rows_per_tile
        zeros16 = jnp.zeros((_L,), jnp.float32)
        isem = (isem0, isem1, isem2)
        gsem = (gsem0, gsem1, gsem2)
        ssem = (ssem0, ssem1, ssem2)

        pltpu.sync_copy(lmax_h, lmax_v)

        def z_wh(r, carry):
            for k in range(_W // _L):
                whel_v[0, r, pl.ds(k * _L, _L)] = zeros16
            return carry
        lax.fori_loop(0, _C, z_wh, 0)
        for m in range(rows_per_tile // _C):
            pltpu.sync_copy(whel_v.at[0], acc_sh.at[pl.ds(r0 + m * _C, _C)])
        rem = rows_per_tile - (rows_per_tile // _C) * _C
        pltpu.sync_copy(
            whel_v.at[0, pl.ds(0, rem)],
            acc_sh.at[pl.ds(r0 + (rows_per_tile // _C) * _C, rem)])
        plsc.subcore_barrier()

        lmax16 = lmax_v[0, pl.ds(0, _L)]

        def issue_idx(j, sl):
            off = base + j * _C
            pltpu.async_copy(rce_h.at[:, pl.ds(off, _C)], rce_v.at[sl],
                             isem[sl])

        def drain_idx(sl):
            pltpu.make_async_copy(
                rce_h.at[:, pl.ds(0, _C)], rce_v.at[sl], isem[sl]).wait()

        def issue_gathers(j, sl):
            pltpu.async_copy(er_h.at[rce_v.at[sl, 1]], er_v.at[sl], gsem[sl])
            pltpu.async_copy(whel_h.at[rce_v.at[sl, 0]], whel_v.at[sl],
                             gsem[sl])

        def drain_gathers(sl):
            pltpu.make_async_copy(
                er_h.at[pl.ds(0, _C)], er_v.at[sl], gsem[sl]).wait()
            pltpu.make_async_copy(
                whel_h.at[pl.ds(0, _C)], whel_v.at[sl], gsem[sl]).wait()

        def drain_scatter(sl):
            pltpu.make_async_copy(
                whel_v.at[sl], acc_sh.at[pl.ds(0, _C)], ssem[sl]).wait()

        def do_chunk(j, sl, nx, th):
            drain_gathers(sl)

            @pl.when(j + 1 < chunks)
            def _():
                drain_idx(nx)

            @pl.when(j >= 2)
            def _():
                drain_scatter(nx)

            @pl.when(j + 1 < chunks)
            def _():
                issue_gathers(j + 1, nx)

            @plsc.parallel_loop(0, _C // _L)
            def group(g):
                sg = pl.ds(g * _L, _L)
                wbits = rce_v[sl, 2, sg]
                ew16 = jnp.maximum(plsc.bitcast(wbits, jnp.float32), 1e-8)
                colS_v[sl, sg] = rce_v[sl, 1, sg]
                for k in range(_L):
                    i = g * _L + k
                    elv = whel_v[sl, i, pl.ds(128, _L)]
                    erv = er_v[sl, i, :]
                    a2 = erv + lmax16
                    mtv = jnp.where(a2 >= 0.0, a2, 0.2 * a2)
                    a = elv + erv
                    ee = jnp.where(a >= 0.0, a, 0.2 * a) - mtv
                    p16 = ew16[k] * jnp.exp(ee)
                    whel_v[sl, i, pl.ds(128, _L)] = p16
                    for hh in range(4):
                        sc = p16[hh]
                        sa = pl.ds(hh * 32, _L)
                        sb = pl.ds(hh * 32 + _L, _L)
                        whel_v[sl, i, sa] = whel_v[sl, i, sa] * sc
                        whel_v[sl, i, sb] = whel_v[sl, i, sb] * sc

            pltpu.async_copy(whel_v.at[sl], acc_sh.at[colS_v.at[sl]],
                             ssem[sl], add=True)

            @pl.when(j + 2 < chunks)
            def _():
                issue_idx(j + 2, th)

        issue_idx(0, 0)
        drain_idx(0)
        issue_gathers(0, 0)
        issue_idx(1, 1)

        def triple(i, carry):
            do_chunk(3 * i, 0, 1, 2)
            do_chunk(3 * i + 1, 1, 2, 0)
            do_chunk(3 * i + 2, 2, 0, 1)
            return carry
        lax.fori_loop(0, chunks // 3, triple, 0)
        for t in range(chunks - (chunks // 3) * 3):
            j = (chunks // 3) * 3 + t
            do_chunk(j, j % 3, (j + 1) % 3, (j + 2) % 3)
        drain_scatter((chunks - 2) % 3)
        drain_scatter((chunks - 1) % 3)

        plsc.subcore_barrier()

        @pl.when(s < _NS - 1)
        def _copy_full():
            sl = pl.ds(r0, rows_per_tile)
            pltpu.sync_copy(acc_sh.at[sl], acc_o.at[c, sl])

        @pl.when(s == _NS - 1)
        def _copy_last():
            sl = pl.ds(r0, last_rows)
            pltpu.sync_copy(acc_sh.at[sl], acc_o.at[c, sl])

    return edge_kernel


def _fin_body(nb, hf, acc_ref, bias_ref, out_ref):
    acc = acc_ref[0] + acc_ref[1]
    feats = acc[:, :hf]
    h = hf // 32
    parts = [jnp.broadcast_to(acc[:, hf + i:hf + i + 1], (nb, 32))
             for i in range(h)]
    denom = jnp.concatenate(parts, axis=1) + 1e-30
    out_ref[...] = feats / denom + bias_ref[...]


def kernel(x, edge_index, edge_weight, W, a_l, a_r, bias):
    n, in_f = x.shape
    h, _, f = W.shape
    e = edge_index.shape[1]
    hf = h * f

    wcat = jnp.transpose(W, (1, 0, 2)).reshape(in_f, hf)
    eye = jnp.eye(h, dtype=jnp.float32)
    al = jnp.reshape(a_l[:, :, 0][:, :, None] * eye[:, None, :], (hf, h))
    ar = jnp.reshape(a_r[:, :, 0][:, :, None] * eye[:, None, :], (hf, h))
    rce = jnp.concatenate(
        [edge_index,
         lax.bitcast_convert_type(edge_weight, jnp.int32)[None, :]], axis=0)

    nb = 2000
    whel, er16, lmax = pl.pallas_call(
        _pre_body,
        grid=(n // nb,),
        in_specs=[
            pl.BlockSpec((nb, in_f), lambda i: (i, 0)),
            pl.BlockSpec((in_f, hf), lambda i: (0, 0)),
            pl.BlockSpec((hf, h), lambda i: (0, 0)),
            pl.BlockSpec((hf, h), lambda i: (0, 0)),
        ],
        out_specs=[
            pl.BlockSpec((nb, _W), lambda i: (i, 0)),
            pl.BlockSpec((nb, _L), lambda i: (i, 0)),
            pl.BlockSpec((1, 128), lambda i: (0, 0)),
        ],
        out_shape=[
            jax.ShapeDtypeStruct((n, _W), jnp.float32),
            jax.ShapeDtypeStruct((n, _L), jnp.float32),
            jax.ShapeDtypeStruct((1, 128), jnp.float32),
        ],
    )(x, wcat, al, ar)

    acc = _make_edge_kernel(n, e)(rce, er16, lmax, whel)

    out = pl.pallas_call(
        functools.partial(_fin_body, nb, hf),
        grid=(n // nb,),
        in_specs=[
            pl.BlockSpec((_NC, nb, _W), lambda i: (0, i, 0)),
            pl.BlockSpec((1, hf), lambda i: (0, 0)),
        ],
        out_specs=pl.BlockSpec((nb, hf), lambda i: (i, 0)),
        out_shape=jax.ShapeDtypeStruct((n, hf), jnp.float32),
    )(acc, bias.reshape(1, hf))
    return out


# speedup vs baseline: 1.0165x; 1.0165x over previous
import functools

import jax
import jax.numpy as jnp
from jax import lax
from jax.experimental import pallas as pl
from jax.experimental.pallas import tpu as pltpu
from jax.experimental.pallas import tpu_sc as plsc

_NC = 2
_NS = 16
_L = 16
_C = 80
_W = 144


def _pre_body(x_ref, wcat_ref, al_ref, ar_ref, whel_ref, er_ref, lmax_ref):
    x = x_ref[...]
    wh = jnp.dot(x, wcat_ref[...], preferred_element_type=jnp.float32)
    el = jnp.dot(wh, al_ref[...], preferred_element_type=jnp.float32)
    er = jnp.dot(wh, ar_ref[...], preferred_element_type=jnp.float32)
    nb = x.shape[0]
    z12 = jnp.zeros((nb, 12), jnp.float32)
    whel_ref[...] = jnp.concatenate([wh, el, z12], axis=1)
    er_ref[...] = jnp.concatenate([er, z12], axis=1)
    lmax = jnp.max(el, axis=0, keepdims=True)
    bm = jnp.concatenate([lmax, jnp.zeros((1, 124), jnp.float32)], axis=1)
    i = pl.program_id(0)

    @pl.when(i == 0)
    def _():
        lmax_ref[...] = bm

    @pl.when(i > 0)
    def _():
        lmax_ref[...] = jnp.maximum(lmax_ref[...], bm)


def _make_edge_kernel(n, e):
    ew_per = e // (_NC * _NS)
    chunks = ew_per // _C
    rows_per_tile = 632
    n_pad = rows_per_tile * _NS
    last_rows = n - (_NS - 1) * rows_per_tile

    mesh = plsc.VectorSubcoreMesh(core_axis_name="c", subcore_axis_name="s")

    @functools.partial(
        pl.kernel,
        out_type=jax.ShapeDtypeStruct((_NC, n, _W), jnp.float32),
        mesh=mesh,
        compiler_params=pltpu.CompilerParams(
            needs_layout_passes=False, use_tc_tiling_on_sc=False),
        scratch_types=[
            pltpu.VMEM((3, 3, _C), jnp.int32),
            pltpu.VMEM((3, _C), jnp.int32),
            pltpu.VMEM((3, _C, _L), jnp.float32),
            pltpu.VMEM((3, _C, _W), jnp.float32),
            pltpu.VMEM((1, 128), jnp.float32),
            pltpu.VMEM_SHARED((n_pad, _W), jnp.float32),
            pltpu.SemaphoreType.DMA,
            pltpu.SemaphoreType.DMA,
            pltpu.SemaphoreType.DMA,
            pltpu.SemaphoreType.DMA,
            pltpu.SemaphoreType.DMA,
            pltpu.SemaphoreType.DMA,
            pltpu.SemaphoreType.DMA,
            pltpu.SemaphoreType.DMA,
            pltpu.SemaphoreType.DMA,
        ],
    )
    def edge_kernel(rce_h, er_h, lmax_h, whel_h, acc_o,
                    rce_v, colS_v, er_v, whel_v, lmax_v, acc_sh,
                    isem0, isem1, isem2, gsem0, gsem1, gsem2,
                    ssem0, ssem1, ssem2):
        c = lax.axis_index("c")
        s = lax.axis_index("s")
        wid = c * _NS + s
        base = wid * ew_per
        r0 = s * rows_per_tile
        zeros16 = jnp.zeros((_L,), jnp.float32)
        isem = (isem0, isem1, isem2)
        gsem = (gsem0, gsem1, gsem2)
        ssem = (ssem0, ssem1, ssem2)

        pltpu.sync_copy(lmax_h, lmax_v)

        def z_wh(r, carry):
            for k in range(_W // _L):
                whel_v[0, r, pl.ds(k * _L, _L)] = zeros16
            return carry
        lax.fori_loop(0, _C, z_wh, 0)
        for m in range(rows_per_tile // _C):
            pltpu.sync_copy(whel_v.at[0], acc_sh.at[pl.ds(r0 + m * _C, _C)])
        rem = rows_per_tile - (rows_per_tile // _C) * _C
        pltpu.sync_copy(
            whel_v.at[0, pl.ds(0, rem)],
            acc_sh.at[pl.ds(r0 + (rows_per_tile // _C) * _C, rem)])
        plsc.subcore_barrier()

        lmax16 = lmax_v[0, pl.ds(0, _L)]

        def issue_idx(j, sl):
            off = base + j * _C
            pltpu.async_copy(rce_h.at[:, pl.ds(off, _C)], rce_v.at[sl],
                             isem[sl])

        def drain_idx(sl):
            pltpu.make_async_copy(
                rce_h.at[:, pl.ds(0, _C)], rce_v.at[sl], isem[sl]).wait()

        def issue_gathers(j, sl):
            pltpu.async_copy(er_h.at[rce_v.at[sl, 1]], er_v.at[sl], gsem[sl])
            pltpu.async_copy(whel_h.at[rce_v.at[sl, 0]], whel_v.at[sl],
                             gsem[sl])

        def drain_gathers(sl):
            pltpu.make_async_copy(
                er_h.at[pl.ds(0, _C)], er_v.at[sl], gsem[sl]).wait()
            pltpu.make_async_copy(
                whel_h.at[pl.ds(0, _C)], whel_v.at[sl], gsem[sl]).wait()

        def drain_scatter(sl):
            pltpu.make_async_copy(
                whel_v.at[sl], acc_sh.at[pl.ds(0, _C)], ssem[sl]).wait()

        def do_chunk(j, sl, nx, th):
            drain_gathers(sl)

            @pl.when(j + 1 < chunks)
            def _():
                drain_idx(nx)

            @pl.when(j >= 2)
            def _():
                drain_scatter(nx)

            @pl.when(j + 1 < chunks)
            def _():
                issue_gathers(j + 1, nx)

            def group(g, carry2):
                sg = pl.ds(g * _L, _L)
                wbits = rce_v[sl, 2, sg]
                ew16 = jnp.maximum(plsc.bitcast(wbits, jnp.float32), 1e-8)
                colS_v[sl, sg] = rce_v[sl, 1, sg]
                for k in range(_L):
                    i = g * _L + k
                    elv = whel_v[sl, i, pl.ds(128, _L)]
                    erv = er_v[sl, i, :]
                    a2 = erv + lmax16
                    mtv = jnp.where(a2 >= 0.0, a2, 0.2 * a2)
                    a = elv + erv
                    ee = jnp.where(a >= 0.0, a, 0.2 * a) - mtv
                    p16 = ew16[k] * jnp.exp(ee)
                    whel_v[sl, i, pl.ds(128, _L)] = p16
                    for hh in range(4):
                        sc = p16[hh]
                        sa = pl.ds(hh * 32, _L)
                        sb = pl.ds(hh * 32 + _L, _L)
                        whel_v[sl, i, sa] = whel_v[sl, i, sa] * sc
                        whel_v[sl, i, sb] = whel_v[sl, i, sb] * sc
                return carry2
            lax.fori_loop(0, _C // _L, group, 0)

            pltpu.async_copy(whel_v.at[sl], acc_sh.at[colS_v.at[sl]],
                             ssem[sl], add=True)

            @pl.when(j + 2 < chunks)
            def _():
                issue_idx(j + 2, th)

        issue_idx(0, 0)
        drain_idx(0)
        issue_gathers(0, 0)
        issue_idx(1, 1)

        def triple(i, carry):
            do_chunk(3 * i, 0, 1, 2)
            do_chunk(3 * i + 1, 1, 2, 0)
            do_chunk(3 * i + 2, 2, 0, 1)
            return carry
        lax.fori_loop(0, chunks // 3, triple, 0)
        for t in range(chunks - (chunks // 3) * 3):
            j = (chunks // 3) * 3 + t
            do_chunk(j, j % 3, (j + 1) % 3, (j + 2) % 3)
        drain_scatter((chunks - 2) % 3)
        drain_scatter((chunks - 1) % 3)

        plsc.subcore_barrier()

        @pl.when(s < _NS - 1)
        def _copy_full():
            sl = pl.ds(r0, rows_per_tile)
            pltpu.sync_copy(acc_sh.at[sl], acc_o.at[c, sl])

        @pl.when(s == _NS - 1)
        def _copy_last():
            sl = pl.ds(r0, last_rows)
            pltpu.sync_copy(acc_sh.at[sl], acc_o.at[c, sl])

    return edge_kernel


def _fin_body(nb, hf, acc_ref, bias_ref, out_ref):
    acc = acc_ref[0] + acc_ref[1]
    feats = acc[:, :hf]
    h = hf // 32
    parts = [jnp.broadcast_to(acc[:, hf + i:hf + i + 1], (nb, 32))
             for i in range(h)]
    denom = jnp.concatenate(parts, axis=1) + 1e-30
    out_ref[...] = feats / denom + bias_ref[...]


def kernel(x, edge_index, edge_weight, W, a_l, a_r, bias):
    n, in_f = x.shape
    h, _, f = W.shape
    e = edge_index.shape[1]
    hf = h * f

    wcat = jnp.transpose(W, (1, 0, 2)).reshape(in_f, hf)
    eye = jnp.eye(h, dtype=jnp.float32)
    al = jnp.reshape(a_l[:, :, 0][:, :, None] * eye[:, None, :], (hf, h))
    ar = jnp.reshape(a_r[:, :, 0][:, :, None] * eye[:, None, :], (hf, h))
    rce = jnp.concatenate(
        [edge_index,
         lax.bitcast_convert_type(edge_weight, jnp.int32)[None, :]], axis=0)

    nb = 2000
    whel, er16, lmax = pl.pallas_call(
        _pre_body,
        grid=(n // nb,),
        in_specs=[
            pl.BlockSpec((nb, in_f), lambda i: (i, 0)),
            pl.BlockSpec((in_f, hf), lambda i: (0, 0)),
            pl.BlockSpec((hf, h), lambda i: (0, 0)),
            pl.BlockSpec((hf, h), lambda i: (0, 0)),
        ],
        out_specs=[
            pl.BlockSpec((nb, _W), lambda i: (i, 0)),
            pl.BlockSpec((nb, _L), lambda i: (i, 0)),
            pl.BlockSpec((1, 128), lambda i: (0, 0)),
        ],
        out_shape=[
            jax.ShapeDtypeStruct((n, _W), jnp.float32),
            jax.ShapeDtypeStruct((n, _L), jnp.float32),
            jax.ShapeDtypeStruct((1, 128), jnp.float32),
        ],
    )(x, wcat, al, ar)

    acc = _make_edge_kernel(n, e)(rce, er16, lmax, whel)

    out = pl.pallas_call(
        functools.partial(_fin_body, nb, hf),
        grid=(n // nb,),
        in_specs=[
            pl.BlockSpec((_NC, nb, _W), lambda i: (0, i, 0)),
            pl.BlockSpec((1, hf), lambda i: (0, 0)),
        ],
        out_specs=pl.BlockSpec((nb, hf), lambda i: (i, 0)),
        out_shape=jax.ShapeDtypeStruct((n, hf), jnp.float32),
    )(acc, bias.reshape(1, hf))
    return out


# speedup vs baseline: 1.0247x; 1.0081x over previous
import functools

import jax
import jax.numpy as jnp
from jax import lax
from jax.experimental import pallas as pl
from jax.experimental.pallas import tpu as pltpu
from jax.experimental.pallas import tpu_sc as plsc

_NC = 2
_NS = 16
_L = 16
_C = 80
_W = 144


def _pre_body(x_ref, wcat_ref, al_ref, ar_ref, whel_ref, er_ref, lmax_ref):
    x = x_ref[...]
    wh = jnp.dot(x, wcat_ref[...], preferred_element_type=jnp.float32)
    el = jnp.dot(wh, al_ref[...], preferred_element_type=jnp.float32)
    er = jnp.dot(wh, ar_ref[...], preferred_element_type=jnp.float32)
    nb = x.shape[0]
    z12 = jnp.zeros((nb, 12), jnp.float32)
    whel_ref[...] = jnp.concatenate([wh, el, z12], axis=1)
    er_ref[...] = jnp.concatenate([er, z12], axis=1)
    lmax = jnp.max(el, axis=0, keepdims=True)
    bm = jnp.concatenate([lmax, jnp.zeros((1, 124), jnp.float32)], axis=1)
    i = pl.program_id(0)

    @pl.when(i == 0)
    def _():
        lmax_ref[...] = bm

    @pl.when(i > 0)
    def _():
        lmax_ref[...] = jnp.maximum(lmax_ref[...], bm)


def _make_edge_kernel(n, e):
    ew_per = e // (_NC * _NS)
    chunks = ew_per // _C
    rows_per_tile = 632
    n_pad = rows_per_tile * _NS
    last_rows = n - (_NS - 1) * rows_per_tile

    mesh = plsc.VectorSubcoreMesh(core_axis_name="c", subcore_axis_name="s")

    @functools.partial(
        pl.kernel,
        out_type=jax.ShapeDtypeStruct((_NC, n, _W), jnp.float32),
        mesh=mesh,
        compiler_params=pltpu.CompilerParams(
            needs_layout_passes=False, use_tc_tiling_on_sc=False),
        scratch_types=[
            pltpu.VMEM((3, 3, _C), jnp.int32),
            pltpu.VMEM((3, _C), jnp.int32),
            pltpu.VMEM((3, _C, _L), jnp.float32),
            pltpu.VMEM((3, _C, _W), jnp.float32),
            pltpu.VMEM((1, 128), jnp.float32),
            pltpu.VMEM_SHARED((n_pad, _W), jnp.float32),
            pltpu.SemaphoreType.DMA,
            pltpu.SemaphoreType.DMA,
            pltpu.SemaphoreType.DMA,
            pltpu.SemaphoreType.DMA,
            pltpu.SemaphoreType.DMA,
            pltpu.SemaphoreType.DMA,
            pltpu.SemaphoreType.DMA,
            pltpu.SemaphoreType.DMA,
            pltpu.SemaphoreType.DMA,
        ],
    )
    def edge_kernel(rce_h, er_h, lmax_h, whel_h, acc_o,
                    rce_v, colS_v, er_v, whel_v, lmax_v, acc_sh,
                    isem0, isem1, isem2, gsem0, gsem1, gsem2,
                    ssem0, ssem1, ssem2):
        c = lax.axis_index("c")
        s = lax.axis_index("s")
        wid = c * _NS + s
        base = wid * ew_per
        r0 = s * rows_per_tile
        zeros16 = jnp.zeros((_L,), jnp.float32)
        isem = (isem0, isem1, isem2)
        gsem = (gsem0, gsem1, gsem2)
        ssem = (ssem0, ssem1, ssem2)

        pltpu.sync_copy(lmax_h, lmax_v)

        def z_wh(r, carry):
            for k in range(_W // _L):
                whel_v[0, r, pl.ds(k * _L, _L)] = zeros16
            return carry
        lax.fori_loop(0, _C, z_wh, 0)
        for m in range(rows_per_tile // _C):
            pltpu.sync_copy(whel_v.at[0], acc_sh.at[pl.ds(r0 + m * _C, _C)])
        rem = rows_per_tile - (rows_per_tile // _C) * _C
        pltpu.sync_copy(
            whel_v.at[0, pl.ds(0, rem)],
            acc_sh.at[pl.ds(r0 + (rows_per_tile // _C) * _C, rem)])
        plsc.subcore_barrier()

        lmax16 = lmax_v[0, pl.ds(0, _L)]

        def issue_idx(j, sl):
            off = base + j * _C
            pltpu.async_copy(rce_h.at[:, pl.ds(off, _C)], rce_v.at[sl],
                             isem[sl])

        def drain_idx(sl):
            pltpu.make_async_copy(
                rce_h.at[:, pl.ds(0, _C)], rce_v.at[sl], isem[sl]).wait()

        def issue_gathers(j, sl):
            pltpu.async_copy(er_h.at[rce_v.at[sl, 1]], er_v.at[sl], gsem[sl])
            pltpu.async_copy(whel_h.at[rce_v.at[sl, 0]], whel_v.at[sl],
                             gsem[sl])

        def drain_gathers(sl):
            pltpu.make_async_copy(
                er_h.at[pl.ds(0, _C)], er_v.at[sl], gsem[sl]).wait()
            pltpu.make_async_copy(
                whel_h.at[pl.ds(0, _C)], whel_v.at[sl], gsem[sl]).wait()

        def drain_scatter(sl):
            pltpu.make_async_copy(
                whel_v.at[sl], acc_sh.at[pl.ds(0, _C)], ssem[sl]).wait()

        def do_chunk(j, sl, nx, th):
            drain_gathers(sl)

            @pl.when(j + 1 < chunks)
            def _():
                drain_idx(nx)

            @pl.when(j >= 2)
            def _():
                drain_scatter(nx)

            @pl.when(j + 1 < chunks)
            def _():
                issue_gathers(j + 1, nx)

            def score(g, carry2):
                sg = pl.ds(g * _L, _L)
                wbits = rce_v[sl, 2, sg]
                ew16 = jnp.maximum(plsc.bitcast(wbits, jnp.float32), 1e-8)
                colS_v[sl, sg] = rce_v[sl, 1, sg]
                for k in range(_L):
                    i = g * _L + k
                    elv = whel_v[sl, i, pl.ds(128, _L)]
                    erv = er_v[sl, i, :]
                    a2 = erv + lmax16
                    mtv = jnp.where(a2 >= 0.0, a2, 0.2 * a2)
                    a = elv + erv
                    ee = jnp.where(a >= 0.0, a, 0.2 * a) - mtv
                    whel_v[sl, i, pl.ds(128, _L)] = ew16[k] * jnp.exp(ee)
                return carry2
            lax.fori_loop(0, _C // _L, score, 0)

            def scale(g, carry2):
                for k in range(_L):
                    i = g * _L + k
                    p16 = whel_v[sl, i, pl.ds(128, _L)]
                    for hh in range(4):
                        sc = p16[hh]
                        sa = pl.ds(hh * 32, _L)
                        sb = pl.ds(hh * 32 + _L, _L)
                        whel_v[sl, i, sa] = whel_v[sl, i, sa] * sc
                        whel_v[sl, i, sb] = whel_v[sl, i, sb] * sc
                return carry2
            lax.fori_loop(0, _C // _L, scale, 0)

            pltpu.async_copy(whel_v.at[sl], acc_sh.at[colS_v.at[sl]],
                             ssem[sl], add=True)

            @pl.when(j + 2 < chunks)
            def _():
                issue_idx(j + 2, th)

        issue_idx(0, 0)
        drain_idx(0)
        issue_gathers(0, 0)
        issue_idx(1, 1)

        def triple(i, carry):
            do_chunk(3 * i, 0, 1, 2)
            do_chunk(3 * i + 1, 1, 2, 0)
            do_chunk(3 * i + 2, 2, 0, 1)
            return carry
        lax.fori_loop(0, chunks // 3, triple, 0)
        for t in range(chunks - (chunks // 3) * 3):
            j = (chunks // 3) * 3 + t
            do_chunk(j, j % 3, (j + 1) % 3, (j + 2) % 3)
        drain_scatter((chunks - 2) % 3)
        drain_scatter((chunks - 1) % 3)

        plsc.subcore_barrier()

        @pl.when(s < _NS - 1)
        def _copy_full():
            sl = pl.ds(r0, rows_per_tile)
            pltpu.sync_copy(acc_sh.at[sl], acc_o.at[c, sl])

        @pl.when(s == _NS - 1)
        def _copy_last():
            sl = pl.ds(r0, last_rows)
            pltpu.sync_copy(acc_sh.at[sl], acc_o.at[c, sl])

    return edge_kernel


def _fin_body(nb, hf, acc_ref, bias_ref, out_ref):
    acc = acc_ref[0] + acc_ref[1]
    feats = acc[:, :hf]
    h = hf // 32
    parts = [jnp.broadcast_to(acc[:, hf + i:hf + i + 1], (nb, 32))
             for i in range(h)]
    denom = jnp.concatenate(parts, axis=1) + 1e-30
    out_ref[...] = feats / denom + bias_ref[...]


def kernel(x, edge_index, edge_weight, W, a_l, a_r, bias):
    n, in_f = x.shape
    h, _, f = W.shape
    e = edge_index.shape[1]
    hf = h * f

    wcat = jnp.transpose(W, (1, 0, 2)).reshape(in_f, hf)
    eye = jnp.eye(h, dtype=jnp.float32)
    al = jnp.reshape(a_l[:, :, 0][:, :, None] * eye[:, None, :], (hf, h))
    ar = jnp.reshape(a_r[:, :, 0][:, :, None] * eye[:, None, :], (hf, h))
    rce = jnp.concatenate(
        [edge_index,
         lax.bitcast_convert_type(edge_weight, jnp.int32)[None, :]], axis=0)

    nb = 2000
    whel, er16, lmax = pl.pallas_call(
        _pre_body,
        grid=(n // nb,),
        in_specs=[
            pl.BlockSpec((nb, in_f), lambda i: (i, 0)),
            pl.BlockSpec((in_f, hf), lambda i: (0, 0)),
            pl.BlockSpec((hf, h), lambda i: (0, 0)),
            pl.BlockSpec((hf, h), lambda i: (0, 0)),
        ],
        out_specs=[
            pl.BlockSpec((nb, _W), lambda i: (i, 0)),
            pl.BlockSpec((nb, _L), lambda i: (i, 0)),
            pl.BlockSpec((1, 128), lambda i: (0, 0)),
        ],
        out_shape=[
            jax.ShapeDtypeStruct((n, _W), jnp.float32),
            jax.ShapeDtypeStruct((n, _L), jnp.float32),
            jax.ShapeDtypeStruct((1, 128), jnp.float32),
        ],
    )(x, wcat, al, ar)

    acc = _make_edge_kernel(n, e)(rce, er16, lmax, whel)

    out = pl.pallas_call(
        functools.partial(_fin_body, nb, hf),
        grid=(n // nb,),
        in_specs=[
            pl.BlockSpec((_NC, nb, _W), lambda i: (0, i, 0)),
            pl.BlockSpec((1, hf), lambda i: (0, 0)),
        ],
        out_specs=pl.BlockSpec((nb, hf), lambda i: (i, 0)),
        out_shape=jax.ShapeDtypeStruct((n, hf), jnp.float32),
    )(acc, bias.reshape(1, hf))
    return out
